# baseline (speedup 1.0000x reference)
import numpy as np

import jax
import jax.numpy as jnp
from jax import lax
from jax.experimental import pallas as pl
from jax.experimental.pallas import tpu as pltpu
from jax.experimental.pallas import tpu_sc as plsc

_L = 12
_F = 2
_BASE_RES = 32
_N = 131072
_LOG2_T = 19
_TMAX = 2 ** _LOG2_T
_MASK = _TMAX - 1
_P1 = np.int32(2654435761 - 2 ** 32)
_P2 = np.int32(805459861)

_RESS = [int(_BASE_RES * (2 ** l)) for l in range(_L)]
_SIZES = [min((r + 1) ** 3, _TMAX) for r in _RESS]
_OFFS = [0]
for _s in _SIZES:
    _OFFS.append(_OFFS[-1] + _s)
_TOTAL = _OFFS[_L]
_N_DENSE = 2

_NC = 2
_NS = 16
_NW = _NC * _NS
_PTS_PER_W = _N // _NW
_CHUNKS = _PTS_PER_W // 16
_NSLOT = 4 * _L
_COLS = 4 * _L * _F


def _splat(s):
    return jnp.broadcast_to(jnp.asarray(s, jnp.int32), (16,))


def _corner_vectors(u0, u1, u2, resf):
    pis = []
    fr = []
    for u in (u0, u1, u2):
        pos = u * resf
        pi = pos.astype(jnp.int32)
        pis.append(pi)
        fr.append(pos - pi.astype(jnp.float32))
    return pis, fr


def _weights8(fr):
    f0, f1, f2 = fr
    g0 = 1.0 - f0
    g1 = 1.0 - f1
    g2 = 1.0 - f2
    a = [g1 * g2, f1 * g2, g1 * f2, f1 * f2]
    ws = []
    for c in range(8):
        d0 = (c >> 0) & 1
        d1 = (c >> 1) & 1
        d2 = (c >> 2) & 1
        ws.append((f0 if d0 else g0) * a[d1 + 2 * d2])
    return ws


def _body(xyzt_hbm, t0, t1, t2, t3, out_hbm,
          xyzt_v, ibuf, wbuf, rows, obuf, sem):
    tables = (t0, t1, t2, t3)
    wid = lax.axis_index("s") * _NC + lax.axis_index("c")
    base = wid * _PTS_PER_W

    pltpu.sync_copy(xyzt_hbm.at[pl.ds(base * 4, _PTS_PER_W * 4)], xyzt_v)

    lanes = lax.iota(jnp.int32, 16)
    lanes4 = lanes * 4
    lanes96 = lanes * _COLS

    def fire(table, s):
        i0 = ibuf.at[pl.ds(s * 256, 128)]
        i1 = ibuf.at[pl.ds(s * 256 + 128, 128)]
        pltpu.async_copy(table.at[i0], rows.at[pl.ds(s * 256, 128)], sem)
        pltpu.async_copy(table.at[i1], rows.at[pl.ds(s * 256 + 128, 128)], sem)

    def chunk_body(g, carry):
        row0 = g * 16
        pbase = lanes4 + row0 * 4
        xv = plsc.load_gather(xyzt_v, [pbase])
        yv = plsc.load_gather(xyzt_v, [pbase + 1])
        zv = plsc.load_gather(xyzt_v, [pbase + 2])
        tv = plsc.load_gather(xyzt_v, [pbase + 3])
        ts = (tv * 2.0 - 1.0) * 0.9
        ux = (xv + 1.0) * 0.5
        uy = (yv + 1.0) * 0.5
        uz = (zv + 1.0) * 0.5
        ut = (ts + 1.0) * 0.5

        plane_coords = ((ux, uy, uz), (ux, uy, ut), (uy, uz, ut), (ux, uz, ut))

        def emit(s, widx_list, w_list):
            ib = _splat(s * 256) + lanes
            wb = _splat(s * 128) + lanes
            for c in range(8):
                plsc.store_scatter(ibuf, [ib + (c * 16)], widx_list[c])
                plsc.store_scatter(ibuf, [ib + (c * 16 + 128)],
                                   widx_list[c] + 1)
                plsc.store_scatter(wbuf, [wb + (c * 16)], w_list[c])

        for p in range(4):
            u0, u1, u2 = plane_coords[p]
            table = tables[p]
            for l in range(_N_DENSE):
                res = _RESS[l]
                r1 = res + 1
                pis, fr = _corner_vectors(u0, u1, u2, float(res))
                bidx = (pis[0] + pis[1] * r1 + pis[2] * (r1 * r1)
                        + _OFFS[l]) * 2
                idxs = []
                for c in range(8):
                    d0 = (c >> 0) & 1
                    d1 = (c >> 1) & 1
                    d2 = (c >> 2) & 1
                    idxs.append(bidx + 2 * (d0 + d1 * r1 + d2 * r1 * r1))
                s = p * _L + l
                emit(s, idxs, _weights8(fr))
                fire(table, s)

            def hbody(l, c2, p=p, u0=u0, u1=u1, u2=u2, table=table):
                res = lax.shift_left(jnp.int32(_BASE_RES), l)
                resf = res.astype(jnp.float32)
                pis, fr = _corner_vectors(u0, u1, u2, resf)
                k1 = pis[1] * _P1
                k2 = pis[2] * _P2
                off2 = 2 * (l * _TMAX + np.int32(_OFFS[2] - 2 * _TMAX))
                idxs = []
                for c in range(8):
                    d0 = (c >> 0) & 1
                    d1 = (c >> 1) & 1
                    d2 = (c >> 2) & 1
                    h0 = pis[0] + d0 if d0 else pis[0]
                    h1 = k1 + _P1 if d1 else k1
                    h2 = k2 + _P2 if d2 else k2
                    h = lax.bitwise_xor(lax.bitwise_xor(h0, h1), h2)
                    m = lax.bitwise_and(h, jnp.int32(_MASK))
                    idxs.append(m * 2 + off2)
                s = p * _L + l
                emit(s, idxs, _weights8(fr))
                fire(table, s)
                return c2

            lax.fori_loop(_N_DENSE, _L, hbody, 0)

        def dbody(i, c2):
            pltpu.make_async_copy(t0.at[ibuf.at[pl.ds(0, 128)]],
                                  rows.at[pl.ds(0, 128)], sem).wait()
            return c2

        lax.fori_loop(0, 2 * _NSLOT, dbody, 0)

        def abody(s, c2):
            rb = _splat(s * 256) + lanes
            wb = _splat(s * 128) + lanes
            acc0 = jnp.zeros((16,), jnp.float32)
            acc1 = jnp.zeros((16,), jnp.float32)
            for c in range(8):
                w = plsc.load_gather(wbuf, [wb + (c * 16)])
                f0 = plsc.load_gather(rows, [rb + (c * 16)])
                f1 = plsc.load_gather(rows, [rb + (c * 16 + 128)])
                acc0 = acc0 + w * f0
                acc1 = acc1 + w * f1
            ob = lanes96 + _splat(s * 2)
            plsc.store_scatter(obuf, [ob], acc0)
            plsc.store_scatter(obuf, [ob + 1], acc1)
            return c2

        lax.fori_loop(0, _NSLOT, abody, 0)

        pltpu.sync_copy(obuf,
                        out_hbm.at[pl.ds((base + row0) * _COLS, 16 * _COLS)])
        return carry

    lax.fori_loop(0, _CHUNKS, chunk_body, 0)


@jax.jit
def _encode(xyzt, t0, t1, t2, t3):
    mesh = plsc.VectorSubcoreMesh(core_axis_name="c", subcore_axis_name="s")
    fn = pl.kernel(
        _body,
        out_type=jax.ShapeDtypeStruct((_N * _COLS,), jnp.float32),
        mesh=mesh,
        compiler_params=pltpu.CompilerParams(needs_layout_passes=False),
        scratch_types=[
            pltpu.VMEM((_PTS_PER_W * 4,), jnp.float32),
            pltpu.VMEM((_NSLOT * 256,), jnp.int32),
            pltpu.VMEM((_NSLOT * 128,), jnp.float32),
            pltpu.VMEM((_NSLOT * 256,), jnp.float32),
            pltpu.VMEM((16 * _COLS,), jnp.float32),
            pltpu.SemaphoreType.DMA,
        ],
    )
    out = fn(xyzt.reshape(-1), t0.reshape(-1), t1.reshape(-1),
             t2.reshape(-1), t3.reshape(-1))
    return out.reshape(_N, _COLS)


def kernel(xyzt, xyz_table, xyt_table, yzt_table, xzt_table):
    return _encode(xyzt, xyz_table, xyt_table, yzt_table, xzt_table)
